# Initial kernel scaffold; baseline (speedup 1.0000x reference)
import jax
import jax.numpy as jnp
from jax.experimental import pallas as pl


def kernel(x, edge_index, Wl1, bl1, Wr1, Wl2, bl2, Wr2, Wl3, bl3, Wr3, Wc, bc):
    raise NotImplementedError("write your pallas kernel here")


# speedup vs baseline: 4.8784x; 4.8784x over previous
import functools

import jax
import jax.numpy as jnp
from jax import lax
from jax.experimental import pallas as pl
from jax.experimental.pallas import tpu as pltpu
from jax.experimental.pallas import tpu_sc as plsc

_N = 10000
_E = 320000
_D = 128
_NC = 2
_NS = 16
_NW = _NC * _NS
_EPW = _E // _NW
_K = 80
_NCHUNK = _EPW // _K
_NPAD = 10240
_RPT = _NPAD // _NS
_BLK = 1000
_GRID = _N // _BLK


def _mesh():
  return plsc.VectorSubcoreMesh(core_axis_name="c", subcore_axis_name="s",
                                num_cores=_NC, num_subcores=_NS)


def _sc_agg_body(table, src, dst, z128, out_agg, src_v, dst_v, rows_v,
                 sh_agg, sem):
  cid = lax.axis_index("c")
  sid = lax.axis_index("s")
  base = (cid * _NS + sid) * _EPW
  r0 = sid * _RPT
  pltpu.sync_copy(z128, sh_agg.at[pl.ds(r0, _RPT)])
  plsc.subcore_barrier()

  def step(c, carry):
    off = base + c * _K
    pltpu.sync_copy(src.at[pl.ds(off, _K)], src_v)
    pltpu.sync_copy(dst.at[pl.ds(off, _K)], dst_v)
    pltpu.async_copy(table.at[src_v], rows_v, sem).wait()
    pltpu.sync_copy(rows_v, sh_agg.at[dst_v], add=True)
    return carry

  lax.fori_loop(0, _NCHUNK, step, 0)
  plsc.subcore_barrier()
  pltpu.sync_copy(sh_agg.at[pl.ds(r0, _RPT)], out_agg.at[cid, pl.ds(r0, _RPT)])


@functools.lru_cache(maxsize=None)
def _make_sc_agg():
  return pl.kernel(
      _sc_agg_body,
      out_type=[jax.ShapeDtypeStruct((_NC, _NPAD, _D), jnp.float32)],
      mesh=_mesh(),
      scratch_types=[
          pltpu.VMEM((_K,), jnp.int32),
          pltpu.VMEM((_K,), jnp.int32),
          pltpu.VMEM((_K, _D), jnp.float32),
          pltpu.VMEM_SHARED((_NPAD, _D), jnp.float32),
          pltpu.SemaphoreType.DMA,
      ],
      compiler_params=pltpu.CompilerParams(needs_layout_passes=False),
  )


def _sc_aux_body(use_c, src, dst, aux, z128, out_aux, src_v, dst_v, aux_v,
                 sh_aux, sem, *extra):
  if use_c:
    invcnt_v, = extra
  cid = lax.axis_index("c")
  sid = lax.axis_index("s")
  base = (cid * _NS + sid) * _EPW
  r0 = sid * _RPT
  pltpu.sync_copy(z128, sh_aux.at[pl.ds(r0, _RPT)])
  if use_c:
    pltpu.sync_copy(aux, invcnt_v)
    pltpu.sync_copy(z128.at[pl.ds(0, _K)], aux_v)
  else:
    pltpu.sync_copy(aux, aux_v)
  plsc.subcore_barrier()

  lanes = lax.iota(jnp.int32, 16)
  col0 = lanes * 0

  def step(c, carry):
    off = base + c * _K
    pltpu.sync_copy(dst.at[pl.ds(off, _K)], dst_v)
    if use_c:
      pltpu.sync_copy(src.at[pl.ds(off, _K)], src_v)
      for t in range(_K // 16):
        dstv = dst_v[pl.ds(16 * t, 16)]
        w = plsc.load_gather(invcnt_v, [dstv])
        plsc.store_scatter(aux_v, [lanes + (16 * t), col0], w)
      pltpu.sync_copy(aux_v, sh_aux.at[src_v], add=True)
    else:
      pltpu.sync_copy(aux_v, sh_aux.at[dst_v], add=True)
    return carry

  lax.fori_loop(0, _NCHUNK, step, 0)
  plsc.subcore_barrier()
  pltpu.sync_copy(sh_aux.at[pl.ds(r0, _RPT)], out_aux.at[cid, pl.ds(r0, _RPT)])


@functools.lru_cache(maxsize=None)
def _make_sc_aux(use_c):
  scratch = [
      pltpu.VMEM((_K,), jnp.int32),
      pltpu.VMEM((_K,), jnp.int32),
      pltpu.VMEM((_K, _D), jnp.float32),
      pltpu.VMEM_SHARED((_NPAD, _D), jnp.float32),
      pltpu.SemaphoreType.DMA,
  ]
  if use_c:
    scratch.append(pltpu.VMEM((_N,), jnp.float32))
  return pl.kernel(
      functools.partial(_sc_aux_body, use_c),
      out_type=[jax.ShapeDtypeStruct((_NC, _NPAD, _D), jnp.float32)],
      mesh=_mesh(),
      scratch_types=scratch,
      compiler_params=pltpu.CompilerParams(needs_layout_passes=False),
  )


def _dot_t(a, b):
  return lax.dot_general(a, b, (((1,), (1,)), ((), ())),
                         preferred_element_type=jnp.float32)


def _tc1_body(p_ref, cnt_ref, x_ref, wl_ref, bl_ref, wr_ref, h_ref, inv16_ref):
  p = p_ref[0] + p_ref[1]
  cnt = cnt_ref[0, :, 0:1] + cnt_ref[1, :, 0:1]
  inv = 1.0 / jnp.maximum(cnt, 1.0)
  z = _dot_t(p * inv, wl_ref[...]) + bl_ref[...] + _dot_t(x_ref[...], wr_ref[...])
  h_ref[...] = jnp.maximum(z, 0.0)
  inv16_ref[...] = jnp.broadcast_to(inv, (_BLK, 16))


def _tc2_body(p_ref, cp_ref, inv16_ref, h1_ref, wl2_ref, bl2_ref, wr2_ref,
              wl3_ref, bl3_ref, wr3_ref, wc_ref, bc_ref, out_ref, acc):
  i = pl.program_id(0)

  @pl.when(i == 0)
  def _():
    acc[...] = jnp.zeros_like(acc)

  p = p_ref[0] + p_ref[1]
  inv = inv16_ref[:, 0:1]
  z = (_dot_t(p * inv, wl2_ref[...]) + bl2_ref[...]
       + _dot_t(h1_ref[...], wr2_ref[...]))
  h2 = jnp.maximum(z, 0.0)
  c = cp_ref[0, :, 0:1] + cp_ref[1, :, 0:1]
  sa = lax.dot_general(c, h2, (((0,), (0,)), ((), ())),
                       preferred_element_type=jnp.float32)
  sh = jnp.sum(h2, axis=0, keepdims=True)
  acc[0:1, :] = acc[0:1, :] + sa
  acc[1:2, :] = acc[1:2, :] + sh

  @pl.when(i == pl.num_programs(0) - 1)
  def _():
    scale = 1.0 / _N
    g = (_dot_t(acc[0:1, :] * scale, wl3_ref[...]) + bl3_ref[...]
         + _dot_t(acc[1:2, :] * scale, wr3_ref[...]))
    out_ref[...] = _dot_t(g, wc_ref[...]) + bc_ref[...]


def _row_spec(shape3):
  return pl.BlockSpec(shape3, lambda i: (0, i, 0))


_tc1 = pl.pallas_call(
    _tc1_body,
    grid=(_GRID,),
    in_specs=[
        _row_spec((_NC, _BLK, _D)),
        _row_spec((_NC, _BLK, _D)),
        pl.BlockSpec((_BLK, _D), lambda i: (i, 0)),
        pl.BlockSpec((_D, _D), lambda i: (0, 0)),
        pl.BlockSpec((1, _D), lambda i: (0, 0)),
        pl.BlockSpec((_D, _D), lambda i: (0, 0)),
    ],
    out_specs=[
        pl.BlockSpec((_BLK, _D), lambda i: (i, 0)),
        pl.BlockSpec((_BLK, 16), lambda i: (i, 0)),
    ],
    out_shape=[
        jax.ShapeDtypeStruct((_N, _D), jnp.float32),
        jax.ShapeDtypeStruct((_N, 16), jnp.float32),
    ],
)

_tc2 = pl.pallas_call(
    _tc2_body,
    grid=(_GRID,),
    in_specs=[
        _row_spec((_NC, _BLK, _D)),
        _row_spec((_NC, _BLK, _D)),
        pl.BlockSpec((_BLK, 16), lambda i: (i, 0)),
        pl.BlockSpec((_BLK, _D), lambda i: (i, 0)),
        pl.BlockSpec((_D, _D), lambda i: (0, 0)),
        pl.BlockSpec((1, _D), lambda i: (0, 0)),
        pl.BlockSpec((_D, _D), lambda i: (0, 0)),
        pl.BlockSpec((_D, _D), lambda i: (0, 0)),
        pl.BlockSpec((1, _D), lambda i: (0, 0)),
        pl.BlockSpec((_D, _D), lambda i: (0, 0)),
        pl.BlockSpec((5, _D), lambda i: (0, 0)),
        pl.BlockSpec((1, 5), lambda i: (0, 0)),
    ],
    out_specs=pl.BlockSpec((1, 5), lambda i: (0, 0)),
    out_shape=jax.ShapeDtypeStruct((1, 5), jnp.float32),
    scratch_shapes=[pltpu.VMEM((8, _D), jnp.float32)],
)


def kernel(x, edge_index, Wl1, bl1, Wr1, Wl2, bl2, Wr2, Wl3, bl3, Wr3, Wc, bc):
  src = edge_index[0]
  dst = edge_index[1]
  ones = jnp.zeros((_K, _D), jnp.float32).at[:, 0].set(1.0)
  z128 = jnp.zeros((_RPT, _D), jnp.float32)

  cnt_p, = _make_sc_aux(False)(src, dst, ones, z128)
  agg_p, = _make_sc_agg()(x, src, dst, z128)
  h1, inv16 = _tc1(agg_p, cnt_p, x, Wl1, bl1.reshape(1, _D), Wr1)
  c_p, = _make_sc_aux(True)(src, dst, inv16[:, 0], z128)
  agg_p2, = _make_sc_agg()(h1, src, dst, z128)
  out = _tc2(agg_p2, c_p, inv16, h1,
             Wl2, bl2.reshape(1, _D), Wr2,
             Wl3, bl3.reshape(1, _D), Wr3,
             Wc, bc.reshape(1, 5))
  return out


# speedup vs baseline: 6.5162x; 1.3357x over previous
import functools

import jax
import jax.numpy as jnp
from jax import lax
from jax.experimental import pallas as pl
from jax.experimental.pallas import tpu as pltpu
from jax.experimental.pallas import tpu_sc as plsc

_N = 10000
_E = 320000
_D = 128
_NC = 2
_NS = 16
_NW = _NC * _NS
_EPW = _E // _NW
_K = 80
_NCHUNK = _EPW // _K
_NPAD = 10240
_RPT = _NPAD // _NS
_BLK = 1000
_GRID = _N // _BLK


def _mesh():
  return plsc.VectorSubcoreMesh(core_axis_name="c", subcore_axis_name="s",
                                num_cores=_NC, num_subcores=_NS)


def _sc_agg_body(table, src, dst, z128, out_agg,
                 src_v0, dst_v0, rows_v0, src_v1, dst_v1, rows_v1,
                 sh_agg, sem0, sem1):
  cid = lax.axis_index("c")
  sid = lax.axis_index("s")
  base = (cid * _NS + sid) * _EPW
  r0 = sid * _RPT
  pltpu.sync_copy(z128, sh_agg.at[pl.ds(r0, _RPT)])
  plsc.subcore_barrier()

  bufs = ((src_v0, dst_v0, rows_v0, sem0), (src_v1, dst_v1, rows_v1, sem1))

  pltpu.sync_copy(src.at[pl.ds(base, _K)], src_v0)
  pltpu.sync_copy(dst.at[pl.ds(base, _K)], dst_v0)
  pltpu.async_copy(table.at[src_v0], rows_v0, sem0)

  def half_step(i, p):
    src_p, dst_p, rows_p, sem_p = bufs[p]
    src_q, dst_q, rows_q, sem_q = bufs[1 - p]

    @pl.when(i < _NCHUNK - 1)
    def _():
      off = base + (i + 1) * _K
      pltpu.sync_copy(src.at[pl.ds(off, _K)], src_q)
      pltpu.sync_copy(dst.at[pl.ds(off, _K)], dst_q)
      pltpu.async_copy(table.at[src_q], rows_q, sem_q)

    pltpu.make_async_copy(table.at[src_p], rows_p, sem_p).wait()
    pltpu.sync_copy(rows_p, sh_agg.at[dst_p], add=True)

  def step(j, carry):
    half_step(2 * j, 0)
    half_step(2 * j + 1, 1)
    return carry

  lax.fori_loop(0, _NCHUNK // 2, step, 0)
  if _NCHUNK % 2:
    half_step(_NCHUNK - 1, 0)
  plsc.subcore_barrier()
  pltpu.sync_copy(sh_agg.at[pl.ds(r0, _RPT)], out_agg.at[cid, pl.ds(r0, _RPT)])


@functools.lru_cache(maxsize=None)
def _make_sc_agg():
  return pl.kernel(
      _sc_agg_body,
      out_type=[jax.ShapeDtypeStruct((_NC, _NPAD, _D), jnp.float32)],
      mesh=_mesh(),
      scratch_types=[
          pltpu.VMEM((_K,), jnp.int32),
          pltpu.VMEM((_K,), jnp.int32),
          pltpu.VMEM((_K, _D), jnp.float32),
          pltpu.VMEM((_K,), jnp.int32),
          pltpu.VMEM((_K,), jnp.int32),
          pltpu.VMEM((_K, _D), jnp.float32),
          pltpu.VMEM_SHARED((_NPAD, _D), jnp.float32),
          pltpu.SemaphoreType.DMA,
          pltpu.SemaphoreType.DMA,
      ],
      compiler_params=pltpu.CompilerParams(needs_layout_passes=False),
  )


def _sc_aux_body(use_c, src, dst, aux, z128, out_aux, src_v, dst_v, aux_v,
                 sh_aux, sem, *extra):
  if use_c:
    invcnt_v, = extra
  cid = lax.axis_index("c")
  sid = lax.axis_index("s")
  base = (cid * _NS + sid) * _EPW
  r0 = sid * _RPT
  pltpu.sync_copy(z128, sh_aux.at[pl.ds(r0, _RPT)])
  if use_c:
    pltpu.sync_copy(aux, invcnt_v)
    pltpu.sync_copy(z128.at[pl.ds(0, _K)], aux_v)
  else:
    pltpu.sync_copy(aux, aux_v)
  plsc.subcore_barrier()

  lanes = lax.iota(jnp.int32, 16)
  col0 = lanes * 0

  def step(c, carry):
    off = base + c * _K
    pltpu.sync_copy(dst.at[pl.ds(off, _K)], dst_v)
    if use_c:
      pltpu.sync_copy(src.at[pl.ds(off, _K)], src_v)
      for t in range(_K // 16):
        dstv = dst_v[pl.ds(16 * t, 16)]
        w = plsc.load_gather(invcnt_v, [dstv])
        plsc.store_scatter(aux_v, [lanes + (16 * t), col0], w)
      pltpu.sync_copy(aux_v, sh_aux.at[src_v], add=True)
    else:
      pltpu.sync_copy(aux_v, sh_aux.at[dst_v], add=True)
    return carry

  lax.fori_loop(0, _NCHUNK, step, 0)
  plsc.subcore_barrier()
  pltpu.sync_copy(sh_aux.at[pl.ds(r0, _RPT)], out_aux.at[cid, pl.ds(r0, _RPT)])


@functools.lru_cache(maxsize=None)
def _make_sc_aux(use_c):
  scratch = [
      pltpu.VMEM((_K,), jnp.int32),
      pltpu.VMEM((_K,), jnp.int32),
      pltpu.VMEM((_K, _D), jnp.float32),
      pltpu.VMEM_SHARED((_NPAD, _D), jnp.float32),
      pltpu.SemaphoreType.DMA,
  ]
  if use_c:
    scratch.append(pltpu.VMEM((_N,), jnp.float32))
  return pl.kernel(
      functools.partial(_sc_aux_body, use_c),
      out_type=[jax.ShapeDtypeStruct((_NC, _NPAD, _D), jnp.float32)],
      mesh=_mesh(),
      scratch_types=scratch,
      compiler_params=pltpu.CompilerParams(needs_layout_passes=False),
  )


def _dot_t(a, b):
  return lax.dot_general(a, b, (((1,), (1,)), ((), ())),
                         preferred_element_type=jnp.float32)


def _tc1_body(p_ref, cnt_ref, x_ref, wl_ref, bl_ref, wr_ref, h_ref, inv16_ref):
  p = p_ref[0] + p_ref[1]
  cnt = cnt_ref[0, :, 0:1] + cnt_ref[1, :, 0:1]
  inv = 1.0 / jnp.maximum(cnt, 1.0)
  z = _dot_t(p * inv, wl_ref[...]) + bl_ref[...] + _dot_t(x_ref[...], wr_ref[...])
  h_ref[...] = jnp.maximum(z, 0.0)
  inv16_ref[...] = jnp.broadcast_to(inv, (_BLK, 16))


def _tc2_body(p_ref, cp_ref, inv16_ref, h1_ref, wl2_ref, bl2_ref, wr2_ref,
              wl3_ref, bl3_ref, wr3_ref, wc_ref, bc_ref, out_ref, acc):
  i = pl.program_id(0)

  @pl.when(i == 0)
  def _():
    acc[...] = jnp.zeros_like(acc)

  p = p_ref[0] + p_ref[1]
  inv = inv16_ref[:, 0:1]
  z = (_dot_t(p * inv, wl2_ref[...]) + bl2_ref[...]
       + _dot_t(h1_ref[...], wr2_ref[...]))
  h2 = jnp.maximum(z, 0.0)
  c = cp_ref[0, :, 0:1] + cp_ref[1, :, 0:1]
  sa = lax.dot_general(c, h2, (((0,), (0,)), ((), ())),
                       preferred_element_type=jnp.float32)
  sh = jnp.sum(h2, axis=0, keepdims=True)
  acc[0:1, :] = acc[0:1, :] + sa
  acc[1:2, :] = acc[1:2, :] + sh

  @pl.when(i == pl.num_programs(0) - 1)
  def _():
    scale = 1.0 / _N
    g = (_dot_t(acc[0:1, :] * scale, wl3_ref[...]) + bl3_ref[...]
         + _dot_t(acc[1:2, :] * scale, wr3_ref[...]))
    out_ref[...] = _dot_t(g, wc_ref[...]) + bc_ref[...]


def _row_spec(shape3):
  return pl.BlockSpec(shape3, lambda i: (0, i, 0))


_tc1 = pl.pallas_call(
    _tc1_body,
    grid=(_GRID,),
    in_specs=[
        _row_spec((_NC, _BLK, _D)),
        _row_spec((_NC, _BLK, _D)),
        pl.BlockSpec((_BLK, _D), lambda i: (i, 0)),
        pl.BlockSpec((_D, _D), lambda i: (0, 0)),
        pl.BlockSpec((1, _D), lambda i: (0, 0)),
        pl.BlockSpec((_D, _D), lambda i: (0, 0)),
    ],
    out_specs=[
        pl.BlockSpec((_BLK, _D), lambda i: (i, 0)),
        pl.BlockSpec((_BLK, 16), lambda i: (i, 0)),
    ],
    out_shape=[
        jax.ShapeDtypeStruct((_N, _D), jnp.float32),
        jax.ShapeDtypeStruct((_N, 16), jnp.float32),
    ],
)

_tc2 = pl.pallas_call(
    _tc2_body,
    grid=(_GRID,),
    in_specs=[
        _row_spec((_NC, _BLK, _D)),
        _row_spec((_NC, _BLK, _D)),
        pl.BlockSpec((_BLK, 16), lambda i: (i, 0)),
        pl.BlockSpec((_BLK, _D), lambda i: (i, 0)),
        pl.BlockSpec((_D, _D), lambda i: (0, 0)),
        pl.BlockSpec((1, _D), lambda i: (0, 0)),
        pl.BlockSpec((_D, _D), lambda i: (0, 0)),
        pl.BlockSpec((_D, _D), lambda i: (0, 0)),
        pl.BlockSpec((1, _D), lambda i: (0, 0)),
        pl.BlockSpec((_D, _D), lambda i: (0, 0)),
        pl.BlockSpec((5, _D), lambda i: (0, 0)),
        pl.BlockSpec((1, 5), lambda i: (0, 0)),
    ],
    out_specs=pl.BlockSpec((1, 5), lambda i: (0, 0)),
    out_shape=jax.ShapeDtypeStruct((1, 5), jnp.float32),
    scratch_shapes=[pltpu.VMEM((8, _D), jnp.float32)],
)


def kernel(x, edge_index, Wl1, bl1, Wr1, Wl2, bl2, Wr2, Wl3, bl3, Wr3, Wc, bc):
  src = edge_index[0]
  dst = edge_index[1]
  ones = jnp.zeros((_K, _D), jnp.float32).at[:, 0].set(1.0)
  z128 = jnp.zeros((_RPT, _D), jnp.float32)

  cnt_p, = _make_sc_aux(False)(src, dst, ones, z128)
  agg_p, = _make_sc_agg()(x, src, dst, z128)
  h1, inv16 = _tc1(agg_p, cnt_p, x, Wl1, bl1.reshape(1, _D), Wr1)
  c_p, = _make_sc_aux(True)(src, dst, inv16[:, 0], z128)
  agg_p2, = _make_sc_agg()(h1, src, dst, z128)
  out = _tc2(agg_p2, c_p, inv16, h1,
             Wl2, bl2.reshape(1, _D), Wr2,
             Wl3, bl3.reshape(1, _D), Wr3,
             Wc, bc.reshape(1, 5))
  return out


# speedup vs baseline: 7.5128x; 1.1529x over previous
import functools

import jax
import jax.numpy as jnp
from jax import lax
from jax.experimental import pallas as pl
from jax.experimental.pallas import tpu as pltpu
from jax.experimental.pallas import tpu_sc as plsc

_N = 10000
_E = 320000
_D = 128
_NC = 2
_NS = 16
_NW = _NC * _NS
_EPW = _E // _NW
_K = 80
_NCHUNK = _EPW // _K
_NPAD = 10240
_RPT = _NPAD // _NS
_BLK = 1000
_GRID = _N // _BLK


def _mesh():
  return plsc.VectorSubcoreMesh(core_axis_name="c", subcore_axis_name="s",
                                num_cores=_NC, num_subcores=_NS)


def _sc_agg_body(table, src, dst, z128, out_agg,
                 src_v0, dst_v0, rows_v0, src_v1, dst_v1, rows_v1,
                 sh_agg, sem0, sem1):
  cid = lax.axis_index("c")
  sid = lax.axis_index("s")
  base = (cid * _NS + sid) * _EPW
  r0 = sid * _RPT
  pltpu.sync_copy(z128, sh_agg.at[pl.ds(r0, _RPT)])
  plsc.subcore_barrier()

  bufs = ((src_v0, dst_v0, rows_v0, sem0), (src_v1, dst_v1, rows_v1, sem1))

  pltpu.sync_copy(src.at[pl.ds(base, _K)], src_v0)
  pltpu.sync_copy(dst.at[pl.ds(base, _K)], dst_v0)
  pltpu.async_copy(table.at[src_v0], rows_v0, sem0)

  def half_step(i, p):
    src_p, dst_p, rows_p, sem_p = bufs[p]
    src_q, dst_q, rows_q, sem_q = bufs[1 - p]

    @pl.when(i < _NCHUNK - 1)
    def _():
      off = base + (i + 1) * _K
      pltpu.sync_copy(src.at[pl.ds(off, _K)], src_q)
      pltpu.sync_copy(dst.at[pl.ds(off, _K)], dst_q)
      pltpu.async_copy(table.at[src_q], rows_q, sem_q)

    pltpu.make_async_copy(table.at[src_p], rows_p, sem_p).wait()
    pltpu.sync_copy(rows_p, sh_agg.at[dst_p], add=True)

  def step(j, carry):
    half_step(2 * j, 0)
    half_step(2 * j + 1, 1)
    return carry

  lax.fori_loop(0, _NCHUNK // 2, step, 0)
  if _NCHUNK % 2:
    half_step(_NCHUNK - 1, 0)
  plsc.subcore_barrier()
  pltpu.sync_copy(sh_agg.at[pl.ds(r0, _RPT)], out_agg.at[cid, pl.ds(r0, _RPT)])


@functools.lru_cache(maxsize=None)
def _make_sc_agg():
  return pl.kernel(
      _sc_agg_body,
      out_type=[jax.ShapeDtypeStruct((_NC, _NPAD, _D), jnp.float32)],
      mesh=_mesh(),
      scratch_types=[
          pltpu.VMEM((_K,), jnp.int32),
          pltpu.VMEM((_K,), jnp.int32),
          pltpu.VMEM((_K, _D), jnp.float32),
          pltpu.VMEM((_K,), jnp.int32),
          pltpu.VMEM((_K,), jnp.int32),
          pltpu.VMEM((_K, _D), jnp.float32),
          pltpu.VMEM_SHARED((_NPAD, _D), jnp.float32),
          pltpu.SemaphoreType.DMA,
          pltpu.SemaphoreType.DMA,
      ],
      compiler_params=pltpu.CompilerParams(needs_layout_passes=False),
  )


def _sc_aux_body(use_c, src, dst, aux, z128, out_aux,
                 src_v0, dst_v0, aux_v0, src_v1, dst_v1, aux_v1,
                 sh_aux, sem0, sem1, *extra):
  if use_c:
    invcnt_v, = extra
  cid = lax.axis_index("c")
  sid = lax.axis_index("s")
  base = (cid * _NS + sid) * _EPW
  r0 = sid * _RPT
  pltpu.sync_copy(z128, sh_aux.at[pl.ds(r0, _RPT)])
  if use_c:
    pltpu.sync_copy(aux, invcnt_v)
    pltpu.sync_copy(z128.at[pl.ds(0, _K)], aux_v0)
    pltpu.sync_copy(z128.at[pl.ds(0, _K)], aux_v1)
  else:
    pltpu.sync_copy(aux, aux_v0)
    pltpu.sync_copy(aux, aux_v1)
  plsc.subcore_barrier()

  lanes = lax.iota(jnp.int32, 16)
  col0 = lanes * 0
  bufs = ((src_v0, dst_v0, aux_v0, sem0), (src_v1, dst_v1, aux_v1, sem1))

  def stage(i, p):
    src_p, dst_p, aux_p, _ = bufs[p]
    off = base + i * _K
    pltpu.sync_copy(dst.at[pl.ds(off, _K)], dst_p)
    if use_c:
      pltpu.sync_copy(src.at[pl.ds(off, _K)], src_p)
      for t in range(_K // 16):
        dstv = dst_p[pl.ds(16 * t, 16)]
        w = plsc.load_gather(invcnt_v, [dstv])
        plsc.store_scatter(aux_p, [lanes + (16 * t), col0], w)

  def issue(p):
    src_p, dst_p, aux_p, sem_p = bufs[p]
    idx = src_p if use_c else dst_p
    pltpu.async_copy(aux_p, sh_aux.at[idx], sem_p, add=True)

  def drain(p):
    src_p, dst_p, aux_p, sem_p = bufs[p]
    idx = src_p if use_c else dst_p
    pltpu.make_async_copy(aux_p, sh_aux.at[idx], sem_p).wait()

  stage(0, 0)

  def half_step(i, p):
    issue(p)

    @pl.when(i < _NCHUNK - 1)
    def _():
      @pl.when(i > 0)
      def _():
        drain(1 - p)
      stage(i + 1, 1 - p)

  def step(j, carry):
    half_step(2 * j, 0)
    half_step(2 * j + 1, 1)
    return carry

  lax.fori_loop(0, _NCHUNK // 2, step, 0)
  if _NCHUNK % 2:
    half_step(_NCHUNK - 1, 0)
  drain(1 - (_NCHUNK - 1) % 2)
  drain((_NCHUNK - 1) % 2)
  plsc.subcore_barrier()
  pltpu.sync_copy(sh_aux.at[pl.ds(r0, _RPT)], out_aux.at[cid, pl.ds(r0, _RPT)])


@functools.lru_cache(maxsize=None)
def _make_sc_aux(use_c):
  scratch = [
      pltpu.VMEM((_K,), jnp.int32),
      pltpu.VMEM((_K,), jnp.int32),
      pltpu.VMEM((_K, _D), jnp.float32),
      pltpu.VMEM((_K,), jnp.int32),
      pltpu.VMEM((_K,), jnp.int32),
      pltpu.VMEM((_K, _D), jnp.float32),
      pltpu.VMEM_SHARED((_NPAD, _D), jnp.float32),
      pltpu.SemaphoreType.DMA,
      pltpu.SemaphoreType.DMA,
  ]
  if use_c:
    scratch.append(pltpu.VMEM((_N,), jnp.float32))
  return pl.kernel(
      functools.partial(_sc_aux_body, use_c),
      out_type=[jax.ShapeDtypeStruct((_NC, _NPAD, _D), jnp.float32)],
      mesh=_mesh(),
      scratch_types=scratch,
      compiler_params=pltpu.CompilerParams(needs_layout_passes=False),
  )


def _dot_t(a, b):
  return lax.dot_general(a, b, (((1,), (1,)), ((), ())),
                         preferred_element_type=jnp.float32)


def _tc1_body(p_ref, cnt_ref, x_ref, wl_ref, bl_ref, wr_ref, h_ref, inv16_ref):
  p = p_ref[0] + p_ref[1]
  cnt = cnt_ref[0, :, 0:1] + cnt_ref[1, :, 0:1]
  inv = 1.0 / jnp.maximum(cnt, 1.0)
  z = _dot_t(p * inv, wl_ref[...]) + bl_ref[...] + _dot_t(x_ref[...], wr_ref[...])
  h_ref[...] = jnp.maximum(z, 0.0)
  inv16_ref[...] = jnp.broadcast_to(inv, (_BLK, 16))


def _tc2_body(p_ref, cp_ref, inv16_ref, h1_ref, wl2_ref, bl2_ref, wr2_ref,
              wl3_ref, bl3_ref, wr3_ref, wc_ref, bc_ref, out_ref, acc):
  i = pl.program_id(0)

  @pl.when(i == 0)
  def _():
    acc[...] = jnp.zeros_like(acc)

  p = p_ref[0] + p_ref[1]
  inv = inv16_ref[:, 0:1]
  z = (_dot_t(p * inv, wl2_ref[...]) + bl2_ref[...]
       + _dot_t(h1_ref[...], wr2_ref[...]))
  h2 = jnp.maximum(z, 0.0)
  c = cp_ref[0, :, 0:1] + cp_ref[1, :, 0:1]
  sa = lax.dot_general(c, h2, (((0,), (0,)), ((), ())),
                       preferred_element_type=jnp.float32)
  sh = jnp.sum(h2, axis=0, keepdims=True)
  acc[0:1, :] = acc[0:1, :] + sa
  acc[1:2, :] = acc[1:2, :] + sh

  @pl.when(i == pl.num_programs(0) - 1)
  def _():
    scale = 1.0 / _N
    g = (_dot_t(acc[0:1, :] * scale, wl3_ref[...]) + bl3_ref[...]
         + _dot_t(acc[1:2, :] * scale, wr3_ref[...]))
    out_ref[...] = _dot_t(g, wc_ref[...]) + bc_ref[...]


def _row_spec(shape3):
  return pl.BlockSpec(shape3, lambda i: (0, i, 0))


_tc1 = pl.pallas_call(
    _tc1_body,
    grid=(_GRID,),
    in_specs=[
        _row_spec((_NC, _BLK, _D)),
        _row_spec((_NC, _BLK, _D)),
        pl.BlockSpec((_BLK, _D), lambda i: (i, 0)),
        pl.BlockSpec((_D, _D), lambda i: (0, 0)),
        pl.BlockSpec((1, _D), lambda i: (0, 0)),
        pl.BlockSpec((_D, _D), lambda i: (0, 0)),
    ],
    out_specs=[
        pl.BlockSpec((_BLK, _D), lambda i: (i, 0)),
        pl.BlockSpec((_BLK, 16), lambda i: (i, 0)),
    ],
    out_shape=[
        jax.ShapeDtypeStruct((_N, _D), jnp.float32),
        jax.ShapeDtypeStruct((_N, 16), jnp.float32),
    ],
)

_tc2 = pl.pallas_call(
    _tc2_body,
    grid=(_GRID,),
    in_specs=[
        _row_spec((_NC, _BLK, _D)),
        _row_spec((_NC, _BLK, _D)),
        pl.BlockSpec((_BLK, 16), lambda i: (i, 0)),
        pl.BlockSpec((_BLK, _D), lambda i: (i, 0)),
        pl.BlockSpec((_D, _D), lambda i: (0, 0)),
        pl.BlockSpec((1, _D), lambda i: (0, 0)),
        pl.BlockSpec((_D, _D), lambda i: (0, 0)),
        pl.BlockSpec((_D, _D), lambda i: (0, 0)),
        pl.BlockSpec((1, _D), lambda i: (0, 0)),
        pl.BlockSpec((_D, _D), lambda i: (0, 0)),
        pl.BlockSpec((5, _D), lambda i: (0, 0)),
        pl.BlockSpec((1, 5), lambda i: (0, 0)),
    ],
    out_specs=pl.BlockSpec((1, 5), lambda i: (0, 0)),
    out_shape=jax.ShapeDtypeStruct((1, 5), jnp.float32),
    scratch_shapes=[pltpu.VMEM((8, _D), jnp.float32)],
)


def kernel(x, edge_index, Wl1, bl1, Wr1, Wl2, bl2, Wr2, Wl3, bl3, Wr3, Wc, bc):
  src = edge_index[0]
  dst = edge_index[1]
  ones = jnp.zeros((_K, _D), jnp.float32).at[:, 0].set(1.0)
  z128 = jnp.zeros((_RPT, _D), jnp.float32)

  cnt_p, = _make_sc_aux(False)(src, dst, ones, z128)
  agg_p, = _make_sc_agg()(x, src, dst, z128)
  h1, inv16 = _tc1(agg_p, cnt_p, x, Wl1, bl1.reshape(1, _D), Wr1)
  c_p, = _make_sc_aux(True)(src, dst, inv16[:, 0], z128)
  agg_p2, = _make_sc_agg()(h1, src, dst, z128)
  out = _tc2(agg_p2, c_p, inv16, h1,
             Wl2, bl2.reshape(1, _D), Wr2,
             Wl3, bl3.reshape(1, _D), Wr3,
             Wc, bc.reshape(1, 5))
  return out


# speedup vs baseline: 8.3738x; 1.1146x over previous
import functools

import jax
import jax.numpy as jnp
from jax import lax
from jax.experimental import pallas as pl
from jax.experimental.pallas import tpu as pltpu
from jax.experimental.pallas import tpu_sc as plsc

_N = 10000
_E = 320000
_D = 128
_NC = 2
_NS = 16
_NW = _NC * _NS
_EPW = _E // _NW
_K = 80
_NCHUNK = _EPW // _K
_KB = 128
_NCB = _EPW // _KB
_KT = _EPW - _NCB * _KB
_NPAD = 10240
_RPT = _NPAD // _NS
_BLK = 1000
_GRID = _N // _BLK


def _mesh():
  return plsc.VectorSubcoreMesh(core_axis_name="c", subcore_axis_name="s",
                                num_cores=_NC, num_subcores=_NS)


def _sc_agg_body(table, src, dst, z128, out_agg,
                 src_v0, dst_v0, rows_v0, src_v1, dst_v1, rows_v1,
                 src_t, dst_t, rows_t, sh_agg, sem0, sem1):
  cid = lax.axis_index("c")
  sid = lax.axis_index("s")
  base = (cid * _NS + sid) * _EPW
  r0 = sid * _RPT
  pltpu.sync_copy(z128, sh_agg.at[pl.ds(r0, _RPT)])
  plsc.subcore_barrier()

  bufs = ((src_v0, dst_v0, rows_v0, sem0), (src_v1, dst_v1, rows_v1, sem1))

  pltpu.sync_copy(src.at[pl.ds(base, _KB)], src_v0)
  pltpu.sync_copy(dst.at[pl.ds(base, _KB)], dst_v0)
  pltpu.async_copy(table.at[src_v0], rows_v0, sem0)

  def half_step(i, p):
    src_p, dst_p, rows_p, sem_p = bufs[p]
    src_q, dst_q, rows_q, sem_q = bufs[1 - p]

    @pl.when(i < _NCB - 1)
    def _():
      off = base + (i + 1) * _KB
      pltpu.sync_copy(src.at[pl.ds(off, _KB)], src_q)
      pltpu.sync_copy(dst.at[pl.ds(off, _KB)], dst_q)
      pltpu.async_copy(table.at[src_q], rows_q, sem_q)

    pltpu.make_async_copy(table.at[src_p], rows_p, sem_p).wait()
    pltpu.sync_copy(rows_p, sh_agg.at[dst_p], add=True)

  def step(j, carry):
    half_step(2 * j, 0)
    half_step(2 * j + 1, 1)
    return carry

  lax.fori_loop(0, _NCB // 2, step, 0)
  toff = base + _NCB * _KB
  pltpu.sync_copy(src.at[pl.ds(toff, _KT)], src_t)
  pltpu.sync_copy(dst.at[pl.ds(toff, _KT)], dst_t)
  pltpu.async_copy(table.at[src_t], rows_t, sem0).wait()
  pltpu.sync_copy(rows_t, sh_agg.at[dst_t], add=True)
  plsc.subcore_barrier()
  pltpu.sync_copy(sh_agg.at[pl.ds(r0, _RPT)], out_agg.at[cid, pl.ds(r0, _RPT)])


@functools.lru_cache(maxsize=None)
def _make_sc_agg():
  return pl.kernel(
      _sc_agg_body,
      out_type=[jax.ShapeDtypeStruct((_NC, _NPAD, _D), jnp.float32)],
      mesh=_mesh(),
      scratch_types=[
          pltpu.VMEM((_KB,), jnp.int32),
          pltpu.VMEM((_KB,), jnp.int32),
          pltpu.VMEM((_KB, _D), jnp.float32),
          pltpu.VMEM((_KB,), jnp.int32),
          pltpu.VMEM((_KB,), jnp.int32),
          pltpu.VMEM((_KB, _D), jnp.float32),
          pltpu.VMEM((_KT,), jnp.int32),
          pltpu.VMEM((_KT,), jnp.int32),
          pltpu.VMEM((_KT, _D), jnp.float32),
          pltpu.VMEM_SHARED((_NPAD, _D), jnp.float32),
          pltpu.SemaphoreType.DMA,
          pltpu.SemaphoreType.DMA,
      ],
      compiler_params=pltpu.CompilerParams(needs_layout_passes=False),
  )


def _sc_aux_body(use_c, src, dst, aux, z128, out_aux,
                 src_v0, dst_v0, aux_v0, src_v1, dst_v1, aux_v1,
                 sh_aux, sem0, sem1, *extra):
  if use_c:
    invcnt_v, = extra
  cid = lax.axis_index("c")
  sid = lax.axis_index("s")
  base = (cid * _NS + sid) * _EPW
  r0 = sid * _RPT
  pltpu.sync_copy(z128, sh_aux.at[pl.ds(r0, _RPT)])
  if use_c:
    pltpu.sync_copy(aux, invcnt_v)
    pltpu.sync_copy(z128.at[pl.ds(0, _K)], aux_v0)
    pltpu.sync_copy(z128.at[pl.ds(0, _K)], aux_v1)
  else:
    pltpu.sync_copy(aux, aux_v0)
    pltpu.sync_copy(aux, aux_v1)
  plsc.subcore_barrier()

  lanes = lax.iota(jnp.int32, 16)
  col0 = lanes * 0
  bufs = ((src_v0, dst_v0, aux_v0, sem0), (src_v1, dst_v1, aux_v1, sem1))

  def stage(i, p):
    src_p, dst_p, aux_p, _ = bufs[p]
    off = base + i * _K
    pltpu.sync_copy(dst.at[pl.ds(off, _K)], dst_p)
    if use_c:
      pltpu.sync_copy(src.at[pl.ds(off, _K)], src_p)
      for t in range(_K // 16):
        dstv = dst_p[pl.ds(16 * t, 16)]
        w = plsc.load_gather(invcnt_v, [dstv])
        plsc.store_scatter(aux_p, [lanes + (16 * t), col0], w)

  def issue(p):
    src_p, dst_p, aux_p, sem_p = bufs[p]
    idx = src_p if use_c else dst_p
    pltpu.async_copy(aux_p, sh_aux.at[idx], sem_p, add=True)

  def drain(p):
    src_p, dst_p, aux_p, sem_p = bufs[p]
    idx = src_p if use_c else dst_p
    pltpu.make_async_copy(aux_p, sh_aux.at[idx], sem_p).wait()

  stage(0, 0)

  def half_step(i, p):
    issue(p)

    @pl.when(i < _NCHUNK - 1)
    def _():
      @pl.when(i > 0)
      def _():
        drain(1 - p)
      stage(i + 1, 1 - p)

  def step(j, carry):
    half_step(2 * j, 0)
    half_step(2 * j + 1, 1)
    return carry

  lax.fori_loop(0, _NCHUNK // 2, step, 0)
  if _NCHUNK % 2:
    half_step(_NCHUNK - 1, 0)
  drain(1 - (_NCHUNK - 1) % 2)
  drain((_NCHUNK - 1) % 2)
  plsc.subcore_barrier()
  pltpu.sync_copy(sh_aux.at[pl.ds(r0, _RPT)], out_aux.at[cid, pl.ds(r0, _RPT)])


@functools.lru_cache(maxsize=None)
def _make_sc_aux(use_c):
  scratch = [
      pltpu.VMEM((_K,), jnp.int32),
      pltpu.VMEM((_K,), jnp.int32),
      pltpu.VMEM((_K, _D), jnp.float32),
      pltpu.VMEM((_K,), jnp.int32),
      pltpu.VMEM((_K,), jnp.int32),
      pltpu.VMEM((_K, _D), jnp.float32),
      pltpu.VMEM_SHARED((_NPAD, _D), jnp.float32),
      pltpu.SemaphoreType.DMA,
      pltpu.SemaphoreType.DMA,
  ]
  if use_c:
    scratch.append(pltpu.VMEM((_N,), jnp.float32))
  return pl.kernel(
      functools.partial(_sc_aux_body, use_c),
      out_type=[jax.ShapeDtypeStruct((_NC, _NPAD, _D), jnp.float32)],
      mesh=_mesh(),
      scratch_types=scratch,
      compiler_params=pltpu.CompilerParams(needs_layout_passes=False),
  )


def _dot_t(a, b):
  return lax.dot_general(a, b, (((1,), (1,)), ((), ())),
                         preferred_element_type=jnp.float32)


def _tc1_body(p_ref, cnt_ref, x_ref, wl_ref, bl_ref, wr_ref, h_ref, inv16_ref):
  p = p_ref[0] + p_ref[1]
  cnt = cnt_ref[0, :, 0:1] + cnt_ref[1, :, 0:1]
  inv = 1.0 / jnp.maximum(cnt, 1.0)
  z = _dot_t(p * inv, wl_ref[...]) + bl_ref[...] + _dot_t(x_ref[...], wr_ref[...])
  h_ref[...] = jnp.maximum(z, 0.0)
  inv16_ref[...] = jnp.broadcast_to(inv, (_BLK, 16))


def _tc2_body(p_ref, cp_ref, inv16_ref, h1_ref, wl2_ref, bl2_ref, wr2_ref,
              wl3_ref, bl3_ref, wr3_ref, wc_ref, bc_ref, out_ref, acc):
  i = pl.program_id(0)

  @pl.when(i == 0)
  def _():
    acc[...] = jnp.zeros_like(acc)

  p = p_ref[0] + p_ref[1]
  inv = inv16_ref[:, 0:1]
  z = (_dot_t(p * inv, wl2_ref[...]) + bl2_ref[...]
       + _dot_t(h1_ref[...], wr2_ref[...]))
  h2 = jnp.maximum(z, 0.0)
  c = cp_ref[0, :, 0:1] + cp_ref[1, :, 0:1]
  sa = lax.dot_general(c, h2, (((0,), (0,)), ((), ())),
                       preferred_element_type=jnp.float32)
  sh = jnp.sum(h2, axis=0, keepdims=True)
  acc[0:1, :] = acc[0:1, :] + sa
  acc[1:2, :] = acc[1:2, :] + sh

  @pl.when(i == pl.num_programs(0) - 1)
  def _():
    scale = 1.0 / _N
    g = (_dot_t(acc[0:1, :] * scale, wl3_ref[...]) + bl3_ref[...]
         + _dot_t(acc[1:2, :] * scale, wr3_ref[...]))
    out_ref[...] = _dot_t(g, wc_ref[...]) + bc_ref[...]


def _row_spec(shape3):
  return pl.BlockSpec(shape3, lambda i: (0, i, 0))


_tc1 = pl.pallas_call(
    _tc1_body,
    grid=(_GRID,),
    in_specs=[
        _row_spec((_NC, _BLK, _D)),
        _row_spec((_NC, _BLK, _D)),
        pl.BlockSpec((_BLK, _D), lambda i: (i, 0)),
        pl.BlockSpec((_D, _D), lambda i: (0, 0)),
        pl.BlockSpec((1, _D), lambda i: (0, 0)),
        pl.BlockSpec((_D, _D), lambda i: (0, 0)),
    ],
    out_specs=[
        pl.BlockSpec((_BLK, _D), lambda i: (i, 0)),
        pl.BlockSpec((_BLK, 16), lambda i: (i, 0)),
    ],
    out_shape=[
        jax.ShapeDtypeStruct((_N, _D), jnp.float32),
        jax.ShapeDtypeStruct((_N, 16), jnp.float32),
    ],
)

_tc2 = pl.pallas_call(
    _tc2_body,
    grid=(_GRID,),
    in_specs=[
        _row_spec((_NC, _BLK, _D)),
        _row_spec((_NC, _BLK, _D)),
        pl.BlockSpec((_BLK, 16), lambda i: (i, 0)),
        pl.BlockSpec((_BLK, _D), lambda i: (i, 0)),
        pl.BlockSpec((_D, _D), lambda i: (0, 0)),
        pl.BlockSpec((1, _D), lambda i: (0, 0)),
        pl.BlockSpec((_D, _D), lambda i: (0, 0)),
        pl.BlockSpec((_D, _D), lambda i: (0, 0)),
        pl.BlockSpec((1, _D), lambda i: (0, 0)),
        pl.BlockSpec((_D, _D), lambda i: (0, 0)),
        pl.BlockSpec((5, _D), lambda i: (0, 0)),
        pl.BlockSpec((1, 5), lambda i: (0, 0)),
    ],
    out_specs=pl.BlockSpec((1, 5), lambda i: (0, 0)),
    out_shape=jax.ShapeDtypeStruct((1, 5), jnp.float32),
    scratch_shapes=[pltpu.VMEM((8, _D), jnp.float32)],
)


def kernel(x, edge_index, Wl1, bl1, Wr1, Wl2, bl2, Wr2, Wl3, bl3, Wr3, Wc, bc):
  src = edge_index[0]
  dst = edge_index[1]
  ones = jnp.zeros((_K, _D), jnp.float32).at[:, 0].set(1.0)
  z128 = jnp.zeros((_RPT, _D), jnp.float32)

  cnt_p, = _make_sc_aux(False)(src, dst, ones, z128)
  agg_p, = _make_sc_agg()(x, src, dst, z128)
  h1, inv16 = _tc1(agg_p, cnt_p, x, Wl1, bl1.reshape(1, _D), Wr1)
  c_p, = _make_sc_aux(True)(src, dst, inv16[:, 0], z128)
  agg_p2, = _make_sc_agg()(h1, src, dst, z128)
  out = _tc2(agg_p2, c_p, inv16, h1,
             Wl2, bl2.reshape(1, _D), Wr2,
             Wl3, bl3.reshape(1, _D), Wr3,
             Wc, bc.reshape(1, 5))
  return out


# speedup vs baseline: 8.8424x; 1.0560x over previous
import functools

import jax
import jax.numpy as jnp
from jax import lax
from jax.experimental import pallas as pl
from jax.experimental.pallas import tpu as pltpu
from jax.experimental.pallas import tpu_sc as plsc

_N = 10000
_E = 320000
_D = 128
_NC = 2
_NS = 16
_NW = _NC * _NS
_EPW = _E // _NW
_K = 80
_NCHUNK = _EPW // _K
_KB = 128
_NCB = _EPW // _KB
_KT = _EPW - _NCB * _KB
_NPAD = 10240
_RPT = _NPAD // _NS
_BLK = 1000
_GRID = _N // _BLK


def _mesh():
  return plsc.VectorSubcoreMesh(core_axis_name="c", subcore_axis_name="s",
                                num_cores=_NC, num_subcores=_NS)


def _sc_agg_body(table, src, dst, z128, out_agg,
                 src_v0, dst_v0, rows_v0, src_v1, dst_v1, rows_v1,
                 src_t, dst_t, rows_t, sh_agg, sem0, sem1):
  cid = lax.axis_index("c")
  sid = lax.axis_index("s")
  base = (cid * _NS + sid) * _EPW
  r0 = sid * _RPT
  pltpu.sync_copy(z128, sh_agg.at[pl.ds(r0, _RPT)])
  plsc.subcore_barrier()

  bufs = ((src_v0, dst_v0, rows_v0, sem0), (src_v1, dst_v1, rows_v1, sem1))

  pltpu.sync_copy(src.at[pl.ds(base, _KB)], src_v0)
  pltpu.sync_copy(dst.at[pl.ds(base, _KB)], dst_v0)
  pltpu.async_copy(table.at[src_v0], rows_v0, sem0)

  def half_step(i, p):
    src_p, dst_p, rows_p, sem_p = bufs[p]
    src_q, dst_q, rows_q, sem_q = bufs[1 - p]

    @pl.when(i < _NCB - 1)
    def _():
      off = base + (i + 1) * _KB
      pltpu.sync_copy(src.at[pl.ds(off, _KB)], src_q)
      pltpu.sync_copy(dst.at[pl.ds(off, _KB)], dst_q)
      pltpu.async_copy(table.at[src_q], rows_q, sem_q)

    pltpu.make_async_copy(table.at[src_p], rows_p, sem_p).wait()
    pltpu.sync_copy(rows_p, sh_agg.at[dst_p], add=True)

  def step(j, carry):
    half_step(2 * j, 0)
    half_step(2 * j + 1, 1)
    return carry

  lax.fori_loop(0, _NCB // 2, step, 0)
  toff = base + _NCB * _KB
  pltpu.sync_copy(src.at[pl.ds(toff, _KT)], src_t)
  pltpu.sync_copy(dst.at[pl.ds(toff, _KT)], dst_t)
  pltpu.async_copy(table.at[src_t], rows_t, sem0).wait()
  pltpu.sync_copy(rows_t, sh_agg.at[dst_t], add=True)
  plsc.subcore_barrier()
  pltpu.sync_copy(sh_agg.at[pl.ds(r0, _RPT)], out_agg.at[cid, pl.ds(r0, _RPT)])


@functools.lru_cache(maxsize=None)
def _make_sc_agg():
  return pl.kernel(
      _sc_agg_body,
      out_type=[jax.ShapeDtypeStruct((_NC, _NPAD, _D), jnp.float32)],
      mesh=_mesh(),
      scratch_types=[
          pltpu.VMEM((_KB,), jnp.int32),
          pltpu.VMEM((_KB,), jnp.int32),
          pltpu.VMEM((_KB, _D), jnp.float32),
          pltpu.VMEM((_KB,), jnp.int32),
          pltpu.VMEM((_KB,), jnp.int32),
          pltpu.VMEM((_KB, _D), jnp.float32),
          pltpu.VMEM((_KT,), jnp.int32),
          pltpu.VMEM((_KT,), jnp.int32),
          pltpu.VMEM((_KT, _D), jnp.float32),
          pltpu.VMEM_SHARED((_NPAD, _D), jnp.float32),
          pltpu.SemaphoreType.DMA,
          pltpu.SemaphoreType.DMA,
      ],
      compiler_params=pltpu.CompilerParams(needs_layout_passes=False),
  )


def _sc_aux_body(use_c, src, dst, aux, z128, out_aux,
                 src_v0, dst_v0, aux_v0, src_v1, dst_v1, aux_v1,
                 src_t, dst_t, sh_aux, sem0, sem1, *extra):
  if use_c:
    invcnt_v, = extra
  cid = lax.axis_index("c")
  sid = lax.axis_index("s")
  base = (cid * _NS + sid) * _EPW
  r0 = sid * _RPT
  pltpu.sync_copy(z128, sh_aux.at[pl.ds(r0, _RPT)])
  if use_c:
    pltpu.sync_copy(aux, invcnt_v)
    pltpu.sync_copy(z128.at[pl.ds(0, _KB)], aux_v0)
    pltpu.sync_copy(z128.at[pl.ds(0, _KB)], aux_v1)
  else:
    pltpu.sync_copy(aux, aux_v0)
    pltpu.sync_copy(aux, aux_v1)
  plsc.subcore_barrier()

  lanes = lax.iota(jnp.int32, 16)
  col0 = lanes * 0
  bufs = ((src_v0, dst_v0, aux_v0, sem0), (src_v1, dst_v1, aux_v1, sem1))

  def stage(i, p):
    src_p, dst_p, aux_p, _ = bufs[p]
    off = base + i * _KB
    pltpu.sync_copy(dst.at[pl.ds(off, _KB)], dst_p)
    if use_c:
      pltpu.sync_copy(src.at[pl.ds(off, _KB)], src_p)
      for t in range(_KB // 16):
        dstv = dst_p[pl.ds(16 * t, 16)]
        w = plsc.load_gather(invcnt_v, [dstv])
        plsc.store_scatter(aux_p, [lanes + (16 * t), col0], w)

  def issue(p):
    src_p, dst_p, aux_p, sem_p = bufs[p]
    idx = src_p if use_c else dst_p
    pltpu.async_copy(aux_p, sh_aux.at[idx], sem_p, add=True)

  def drain(p):
    src_p, dst_p, aux_p, sem_p = bufs[p]
    idx = src_p if use_c else dst_p
    pltpu.make_async_copy(aux_p, sh_aux.at[idx], sem_p).wait()

  stage(0, 0)

  def half_step(i, p):
    issue(p)

    @pl.when(i < _NCB - 1)
    def _():
      @pl.when(i > 0)
      def _():
        drain(1 - p)
      stage(i + 1, 1 - p)

  def step(j, carry):
    half_step(2 * j, 0)
    half_step(2 * j + 1, 1)
    return carry

  lax.fori_loop(0, _NCB // 2, step, 0)
  drain(0)
  drain(1)
  toff = base + _NCB * _KB
  pltpu.sync_copy(dst.at[pl.ds(toff, _KT)], dst_t)
  if use_c:
    pltpu.sync_copy(src.at[pl.ds(toff, _KT)], src_t)
    dstv = dst_t[...]
    w = plsc.load_gather(invcnt_v, [dstv])
    plsc.store_scatter(aux_v0, [lanes, col0], w)
    pltpu.sync_copy(aux_v0.at[pl.ds(0, _KT)], sh_aux.at[src_t], add=True)
  else:
    pltpu.sync_copy(aux_v0.at[pl.ds(0, _KT)], sh_aux.at[dst_t], add=True)
  plsc.subcore_barrier()
  pltpu.sync_copy(sh_aux.at[pl.ds(r0, _RPT)], out_aux.at[cid, pl.ds(r0, _RPT)])


@functools.lru_cache(maxsize=None)
def _make_sc_aux(use_c):
  scratch = [
      pltpu.VMEM((_KB,), jnp.int32),
      pltpu.VMEM((_KB,), jnp.int32),
      pltpu.VMEM((_KB, _D), jnp.float32),
      pltpu.VMEM((_KB,), jnp.int32),
      pltpu.VMEM((_KB,), jnp.int32),
      pltpu.VMEM((_KB, _D), jnp.float32),
      pltpu.VMEM((_KT,), jnp.int32),
      pltpu.VMEM((_KT,), jnp.int32),
      pltpu.VMEM_SHARED((_NPAD, _D), jnp.float32),
      pltpu.SemaphoreType.DMA,
      pltpu.SemaphoreType.DMA,
  ]
  if use_c:
    scratch.append(pltpu.VMEM((_N,), jnp.float32))
  return pl.kernel(
      functools.partial(_sc_aux_body, use_c),
      out_type=[jax.ShapeDtypeStruct((_NC, _NPAD, _D), jnp.float32)],
      mesh=_mesh(),
      scratch_types=scratch,
      compiler_params=pltpu.CompilerParams(needs_layout_passes=False),
  )


def _dot_t(a, b):
  return lax.dot_general(a, b, (((1,), (1,)), ((), ())),
                         preferred_element_type=jnp.float32)


def _tc1_body(p_ref, cnt_ref, x_ref, wl_ref, bl_ref, wr_ref, h_ref, inv16_ref):
  p = p_ref[0] + p_ref[1]
  cnt = cnt_ref[0, :, 0:1] + cnt_ref[1, :, 0:1]
  inv = 1.0 / jnp.maximum(cnt, 1.0)
  z = _dot_t(p * inv, wl_ref[...]) + bl_ref[...] + _dot_t(x_ref[...], wr_ref[...])
  h_ref[...] = jnp.maximum(z, 0.0)
  inv16_ref[...] = jnp.broadcast_to(inv, (_BLK, 16))


def _tc2_body(p_ref, cp_ref, inv16_ref, h1_ref, wl2_ref, bl2_ref, wr2_ref,
              wl3_ref, bl3_ref, wr3_ref, wc_ref, bc_ref, out_ref, acc):
  i = pl.program_id(0)

  @pl.when(i == 0)
  def _():
    acc[...] = jnp.zeros_like(acc)

  p = p_ref[0] + p_ref[1]
  inv = inv16_ref[:, 0:1]
  z = (_dot_t(p * inv, wl2_ref[...]) + bl2_ref[...]
       + _dot_t(h1_ref[...], wr2_ref[...]))
  h2 = jnp.maximum(z, 0.0)
  c = cp_ref[0, :, 0:1] + cp_ref[1, :, 0:1]
  sa = lax.dot_general(c, h2, (((0,), (0,)), ((), ())),
                       preferred_element_type=jnp.float32)
  sh = jnp.sum(h2, axis=0, keepdims=True)
  acc[0:1, :] = acc[0:1, :] + sa
  acc[1:2, :] = acc[1:2, :] + sh

  @pl.when(i == pl.num_programs(0) - 1)
  def _():
    scale = 1.0 / _N
    g = (_dot_t(acc[0:1, :] * scale, wl3_ref[...]) + bl3_ref[...]
         + _dot_t(acc[1:2, :] * scale, wr3_ref[...]))
    out_ref[...] = _dot_t(g, wc_ref[...]) + bc_ref[...]


def _row_spec(shape3):
  return pl.BlockSpec(shape3, lambda i: (0, i, 0))


_tc1 = pl.pallas_call(
    _tc1_body,
    grid=(_GRID,),
    in_specs=[
        _row_spec((_NC, _BLK, _D)),
        _row_spec((_NC, _BLK, _D)),
        pl.BlockSpec((_BLK, _D), lambda i: (i, 0)),
        pl.BlockSpec((_D, _D), lambda i: (0, 0)),
        pl.BlockSpec((1, _D), lambda i: (0, 0)),
        pl.BlockSpec((_D, _D), lambda i: (0, 0)),
    ],
    out_specs=[
        pl.BlockSpec((_BLK, _D), lambda i: (i, 0)),
        pl.BlockSpec((_BLK, 16), lambda i: (i, 0)),
    ],
    out_shape=[
        jax.ShapeDtypeStruct((_N, _D), jnp.float32),
        jax.ShapeDtypeStruct((_N, 16), jnp.float32),
    ],
)

_tc2 = pl.pallas_call(
    _tc2_body,
    grid=(_GRID,),
    in_specs=[
        _row_spec((_NC, _BLK, _D)),
        _row_spec((_NC, _BLK, _D)),
        pl.BlockSpec((_BLK, 16), lambda i: (i, 0)),
        pl.BlockSpec((_BLK, _D), lambda i: (i, 0)),
        pl.BlockSpec((_D, _D), lambda i: (0, 0)),
        pl.BlockSpec((1, _D), lambda i: (0, 0)),
        pl.BlockSpec((_D, _D), lambda i: (0, 0)),
        pl.BlockSpec((_D, _D), lambda i: (0, 0)),
        pl.BlockSpec((1, _D), lambda i: (0, 0)),
        pl.BlockSpec((_D, _D), lambda i: (0, 0)),
        pl.BlockSpec((5, _D), lambda i: (0, 0)),
        pl.BlockSpec((1, 5), lambda i: (0, 0)),
    ],
    out_specs=pl.BlockSpec((1, 5), lambda i: (0, 0)),
    out_shape=jax.ShapeDtypeStruct((1, 5), jnp.float32),
    scratch_shapes=[pltpu.VMEM((8, _D), jnp.float32)],
)


def kernel(x, edge_index, Wl1, bl1, Wr1, Wl2, bl2, Wr2, Wl3, bl3, Wr3, Wc, bc):
  src = edge_index[0]
  dst = edge_index[1]
  ones = jnp.zeros((_KB, _D), jnp.float32).at[:, 0].set(1.0)
  z128 = jnp.zeros((_RPT, _D), jnp.float32)

  cnt_p, = _make_sc_aux(False)(src, dst, ones, z128)
  agg_p, = _make_sc_agg()(x, src, dst, z128)
  h1, inv16 = _tc1(agg_p, cnt_p, x, Wl1, bl1.reshape(1, _D), Wr1)
  c_p, = _make_sc_aux(True)(src, dst, inv16[:, 0], z128)
  agg_p2, = _make_sc_agg()(h1, src, dst, z128)
  out = _tc2(agg_p2, c_p, inv16, h1,
             Wl2, bl2.reshape(1, _D), Wr2,
             Wl3, bl3.reshape(1, _D), Wr3,
             Wc, bc.reshape(1, 5))
  return out


# speedup vs baseline: 9.7765x; 1.1056x over previous
import functools

import jax
import jax.numpy as jnp
from jax import lax
from jax.experimental import pallas as pl
from jax.experimental.pallas import tpu as pltpu
from jax.experimental.pallas import tpu_sc as plsc

_N = 10000
_E = 320000
_D = 128
_NC = 2
_NS = 16
_NW = _NC * _NS
_EPW = _E // _NW
_K = 80
_NCHUNK = _EPW // _K
_KB = 128
_NCB = _EPW // _KB
_KT = _EPW - _NCB * _KB
_NPAD = 10240
_RPT = _NPAD // _NS
_BLK = 1000
_GRID = _N // _BLK


def _mesh():
  return plsc.VectorSubcoreMesh(core_axis_name="c", subcore_axis_name="s",
                                num_cores=_NC, num_subcores=_NS)


def _sc_agg_body(table, src, dst, z128, out_agg,
                 src_v0, dst_v0, rows_v0, src_v1, dst_v1, rows_v1,
                 src_t, dst_t, rows_t, sh_agg, sem0, sem1, semi0, semi1):
  cid = lax.axis_index("c")
  sid = lax.axis_index("s")
  base = (cid * _NS + sid) * _EPW
  r0 = sid * _RPT
  pltpu.sync_copy(z128, sh_agg.at[pl.ds(r0, _RPT)])
  plsc.subcore_barrier()

  bufs = ((src_v0, dst_v0, rows_v0, sem0, semi0),
          (src_v1, dst_v1, rows_v1, sem1, semi1))

  def idx_start(i, p):
    src_p, dst_p, _, _, semi_p = bufs[p]
    off = base + i * _KB
    pltpu.async_copy(src.at[pl.ds(off, _KB)], src_p, semi_p)
    pltpu.async_copy(dst.at[pl.ds(off, _KB)], dst_p, semi_p)

  def idx_wait(p):
    src_p, dst_p, _, _, semi_p = bufs[p]
    pltpu.make_async_copy(src.at[pl.ds(base, _KB)], src_p, semi_p).wait()
    pltpu.make_async_copy(dst.at[pl.ds(base, _KB)], dst_p, semi_p).wait()

  idx_start(0, 0)
  idx_wait(0)
  pltpu.async_copy(table.at[src_v0], rows_v0, sem0)
  idx_start(1, 1)

  def half_step(i, p):
    src_p, dst_p, rows_p, sem_p, _ = bufs[p]
    src_q, dst_q, rows_q, sem_q, _ = bufs[1 - p]

    @pl.when(i < _NCB - 1)
    def _():
      idx_wait(1 - p)
      pltpu.async_copy(table.at[src_q], rows_q, sem_q)

    pltpu.make_async_copy(table.at[src_p], rows_p, sem_p).wait()
    pltpu.sync_copy(rows_p, sh_agg.at[dst_p], add=True)

    @pl.when(i < _NCB - 2)
    def _():
      idx_start(i + 2, p)

  def step(j, carry):
    half_step(2 * j, 0)
    half_step(2 * j + 1, 1)
    return carry

  lax.fori_loop(0, _NCB // 2, step, 0)
  toff = base + _NCB * _KB
  pltpu.sync_copy(src.at[pl.ds(toff, _KT)], src_t)
  pltpu.sync_copy(dst.at[pl.ds(toff, _KT)], dst_t)
  pltpu.async_copy(table.at[src_t], rows_t, sem0).wait()
  pltpu.sync_copy(rows_t, sh_agg.at[dst_t], add=True)
  plsc.subcore_barrier()
  pltpu.sync_copy(sh_agg.at[pl.ds(r0, _RPT)], out_agg.at[cid, pl.ds(r0, _RPT)])


@functools.lru_cache(maxsize=None)
def _make_sc_agg():
  return pl.kernel(
      _sc_agg_body,
      out_type=[jax.ShapeDtypeStruct((_NC, _NPAD, _D), jnp.float32)],
      mesh=_mesh(),
      scratch_types=[
          pltpu.VMEM((_KB,), jnp.int32),
          pltpu.VMEM((_KB,), jnp.int32),
          pltpu.VMEM((_KB, _D), jnp.float32),
          pltpu.VMEM((_KB,), jnp.int32),
          pltpu.VMEM((_KB,), jnp.int32),
          pltpu.VMEM((_KB, _D), jnp.float32),
          pltpu.VMEM((_KT,), jnp.int32),
          pltpu.VMEM((_KT,), jnp.int32),
          pltpu.VMEM((_KT, _D), jnp.float32),
          pltpu.VMEM_SHARED((_NPAD, _D), jnp.float32),
          pltpu.SemaphoreType.DMA,
          pltpu.SemaphoreType.DMA,
          pltpu.SemaphoreType.DMA,
          pltpu.SemaphoreType.DMA,
      ],
      compiler_params=pltpu.CompilerParams(needs_layout_passes=False),
  )


def _sc_aux_body(use_c, src, dst, aux, z128, out_aux,
                 src_v0, dst_v0, aux_v0, src_v1, dst_v1, aux_v1,
                 src_t, dst_t, sh_aux, sem0, sem1, *extra):
  if use_c:
    invcnt_v, = extra
  cid = lax.axis_index("c")
  sid = lax.axis_index("s")
  base = (cid * _NS + sid) * _EPW
  r0 = sid * _RPT
  pltpu.sync_copy(z128, sh_aux.at[pl.ds(r0, _RPT)])
  if use_c:
    pltpu.sync_copy(aux, invcnt_v)
    pltpu.sync_copy(z128.at[pl.ds(0, _KB)], aux_v0)
    pltpu.sync_copy(z128.at[pl.ds(0, _KB)], aux_v1)
  else:
    pltpu.sync_copy(aux, aux_v0)
    pltpu.sync_copy(aux, aux_v1)
  plsc.subcore_barrier()

  lanes = lax.iota(jnp.int32, 16)
  col0 = lanes * 0
  bufs = ((src_v0, dst_v0, aux_v0, sem0), (src_v1, dst_v1, aux_v1, sem1))

  def stage(i, p):
    src_p, dst_p, aux_p, _ = bufs[p]
    off = base + i * _KB
    pltpu.sync_copy(dst.at[pl.ds(off, _KB)], dst_p)
    if use_c:
      pltpu.sync_copy(src.at[pl.ds(off, _KB)], src_p)
      for t in range(_KB // 16):
        dstv = dst_p[pl.ds(16 * t, 16)]
        w = plsc.load_gather(invcnt_v, [dstv])
        plsc.store_scatter(aux_p, [lanes + (16 * t), col0], w)

  def issue(p):
    src_p, dst_p, aux_p, sem_p = bufs[p]
    idx = src_p if use_c else dst_p
    pltpu.async_copy(aux_p, sh_aux.at[idx], sem_p, add=True)

  def drain(p):
    src_p, dst_p, aux_p, sem_p = bufs[p]
    idx = src_p if use_c else dst_p
    pltpu.make_async_copy(aux_p, sh_aux.at[idx], sem_p).wait()

  stage(0, 0)

  def half_step(i, p):
    issue(p)

    @pl.when(i < _NCB - 1)
    def _():
      @pl.when(i > 0)
      def _():
        drain(1 - p)
      stage(i + 1, 1 - p)

  def step(j, carry):
    half_step(2 * j, 0)
    half_step(2 * j + 1, 1)
    return carry

  lax.fori_loop(0, _NCB // 2, step, 0)
  drain(0)
  drain(1)
  toff = base + _NCB * _KB
  pltpu.sync_copy(dst.at[pl.ds(toff, _KT)], dst_t)
  if use_c:
    pltpu.sync_copy(src.at[pl.ds(toff, _KT)], src_t)
    dstv = dst_t[...]
    w = plsc.load_gather(invcnt_v, [dstv])
    plsc.store_scatter(aux_v0, [lanes, col0], w)
    pltpu.sync_copy(aux_v0.at[pl.ds(0, _KT)], sh_aux.at[src_t], add=True)
  else:
    pltpu.sync_copy(aux_v0.at[pl.ds(0, _KT)], sh_aux.at[dst_t], add=True)
  plsc.subcore_barrier()
  pltpu.sync_copy(sh_aux.at[pl.ds(r0, _RPT)], out_aux.at[cid, pl.ds(r0, _RPT)])


@functools.lru_cache(maxsize=None)
def _make_sc_aux(use_c):
  scratch = [
      pltpu.VMEM((_KB,), jnp.int32),
      pltpu.VMEM((_KB,), jnp.int32),
      pltpu.VMEM((_KB, _D), jnp.float32),
      pltpu.VMEM((_KB,), jnp.int32),
      pltpu.VMEM((_KB,), jnp.int32),
      pltpu.VMEM((_KB, _D), jnp.float32),
      pltpu.VMEM((_KT,), jnp.int32),
      pltpu.VMEM((_KT,), jnp.int32),
      pltpu.VMEM_SHARED((_NPAD, _D), jnp.float32),
      pltpu.SemaphoreType.DMA,
      pltpu.SemaphoreType.DMA,
  ]
  if use_c:
    scratch.append(pltpu.VMEM((_N,), jnp.float32))
  return pl.kernel(
      functools.partial(_sc_aux_body, use_c),
      out_type=[jax.ShapeDtypeStruct((_NC, _NPAD, _D), jnp.float32)],
      mesh=_mesh(),
      scratch_types=scratch,
      compiler_params=pltpu.CompilerParams(needs_layout_passes=False),
  )


def _dot_t(a, b):
  return lax.dot_general(a, b, (((1,), (1,)), ((), ())),
                         preferred_element_type=jnp.float32)


def _tc1_body(p_ref, cnt_ref, x_ref, wl_ref, bl_ref, wr_ref, h_ref, inv16_ref):
  p = p_ref[0] + p_ref[1]
  cnt = cnt_ref[0, :, 0:1] + cnt_ref[1, :, 0:1]
  inv = 1.0 / jnp.maximum(cnt, 1.0)
  z = _dot_t(p * inv, wl_ref[...]) + bl_ref[...] + _dot_t(x_ref[...], wr_ref[...])
  h_ref[...] = jnp.maximum(z, 0.0)
  inv16_ref[...] = jnp.broadcast_to(inv, (_BLK, 16))


def _tc2_body(p_ref, cp_ref, inv16_ref, h1_ref, wl2_ref, bl2_ref, wr2_ref,
              wl3_ref, bl3_ref, wr3_ref, wc_ref, bc_ref, out_ref, acc):
  i = pl.program_id(0)

  @pl.when(i == 0)
  def _():
    acc[...] = jnp.zeros_like(acc)

  p = p_ref[0] + p_ref[1]
  inv = inv16_ref[:, 0:1]
  z = (_dot_t(p * inv, wl2_ref[...]) + bl2_ref[...]
       + _dot_t(h1_ref[...], wr2_ref[...]))
  h2 = jnp.maximum(z, 0.0)
  c = cp_ref[0, :, 0:1] + cp_ref[1, :, 0:1]
  sa = lax.dot_general(c, h2, (((0,), (0,)), ((), ())),
                       preferred_element_type=jnp.float32)
  sh = jnp.sum(h2, axis=0, keepdims=True)
  acc[0:1, :] = acc[0:1, :] + sa
  acc[1:2, :] = acc[1:2, :] + sh

  @pl.when(i == pl.num_programs(0) - 1)
  def _():
    scale = 1.0 / _N
    g = (_dot_t(acc[0:1, :] * scale, wl3_ref[...]) + bl3_ref[...]
         + _dot_t(acc[1:2, :] * scale, wr3_ref[...]))
    out_ref[...] = _dot_t(g, wc_ref[...]) + bc_ref[...]


def _row_spec(shape3):
  return pl.BlockSpec(shape3, lambda i: (0, i, 0))


_tc1 = pl.pallas_call(
    _tc1_body,
    grid=(_GRID,),
    in_specs=[
        _row_spec((_NC, _BLK, _D)),
        _row_spec((_NC, _BLK, _D)),
        pl.BlockSpec((_BLK, _D), lambda i: (i, 0)),
        pl.BlockSpec((_D, _D), lambda i: (0, 0)),
        pl.BlockSpec((1, _D), lambda i: (0, 0)),
        pl.BlockSpec((_D, _D), lambda i: (0, 0)),
    ],
    out_specs=[
        pl.BlockSpec((_BLK, _D), lambda i: (i, 0)),
        pl.BlockSpec((_BLK, 16), lambda i: (i, 0)),
    ],
    out_shape=[
        jax.ShapeDtypeStruct((_N, _D), jnp.float32),
        jax.ShapeDtypeStruct((_N, 16), jnp.float32),
    ],
)

_tc2 = pl.pallas_call(
    _tc2_body,
    grid=(_GRID,),
    in_specs=[
        _row_spec((_NC, _BLK, _D)),
        _row_spec((_NC, _BLK, _D)),
        pl.BlockSpec((_BLK, 16), lambda i: (i, 0)),
        pl.BlockSpec((_BLK, _D), lambda i: (i, 0)),
        pl.BlockSpec((_D, _D), lambda i: (0, 0)),
        pl.BlockSpec((1, _D), lambda i: (0, 0)),
        pl.BlockSpec((_D, _D), lambda i: (0, 0)),
        pl.BlockSpec((_D, _D), lambda i: (0, 0)),
        pl.BlockSpec((1, _D), lambda i: (0, 0)),
        pl.BlockSpec((_D, _D), lambda i: (0, 0)),
        pl.BlockSpec((5, _D), lambda i: (0, 0)),
        pl.BlockSpec((1, 5), lambda i: (0, 0)),
    ],
    out_specs=pl.BlockSpec((1, 5), lambda i: (0, 0)),
    out_shape=jax.ShapeDtypeStruct((1, 5), jnp.float32),
    scratch_shapes=[pltpu.VMEM((8, _D), jnp.float32)],
)


def kernel(x, edge_index, Wl1, bl1, Wr1, Wl2, bl2, Wr2, Wl3, bl3, Wr3, Wc, bc):
  src = edge_index[0]
  dst = edge_index[1]
  ones = jnp.zeros((_KB, _D), jnp.float32).at[:, 0].set(1.0)
  z128 = jnp.zeros((_RPT, _D), jnp.float32)

  cnt_p, = _make_sc_aux(False)(src, dst, ones, z128)
  agg_p, = _make_sc_agg()(x, src, dst, z128)
  h1, inv16 = _tc1(agg_p, cnt_p, x, Wl1, bl1.reshape(1, _D), Wr1)
  c_p, = _make_sc_aux(True)(src, dst, inv16[:, 0], z128)
  agg_p2, = _make_sc_agg()(h1, src, dst, z128)
  out = _tc2(agg_p2, c_p, inv16, h1,
             Wl2, bl2.reshape(1, _D), Wr2,
             Wl3, bl3.reshape(1, _D), Wr3,
             Wc, bc.reshape(1, 5))
  return out
